# Initial kernel scaffold; baseline (speedup 1.0000x reference)
import jax
import jax.numpy as jnp
from jax.experimental import pallas as pl


def kernel(x, union_features, rel_pair_idxs, obj_unary_w, obj_unary_b, edge_unary_w, edge_unary_b, node_w_ih, node_w_hh, node_b_ih, node_b_hh, edge_w_ih, edge_w_hh, edge_b_ih, edge_b_hh, sub_w, sub_b, obj_w, obj_b, out_w, out_b, in_w, in_b, obj_fc_w, obj_fc_b, rel_fc_w, rel_fc_b):
    raise NotImplementedError("write your pallas kernel here")


# speedup vs baseline: 3.4632x; 3.4632x over previous
import functools

import jax
import jax.numpy as jnp
from jax import lax
from jax.experimental import pallas as pl
from jax.experimental.pallas import tpu as pltpu
from jax.experimental.pallas import tpu_sc as plsc

HID = 128
N_OBJ = 2048
N_REL = 16384
NUM_OBJ_CLS = 151
NUM_REL_CLS = 51

NC = 2
NS = 16
NW = NC * NS
EPW = N_REL // NW


def _sigmoid(x):
    return jax.nn.sigmoid(x)


def _gru_tail(gi, gh, h):
    r = _sigmoid(gi[:, :HID] + gh[:, :HID])
    z = _sigmoid(gi[:, HID:2 * HID] + gh[:, HID:2 * HID])
    n = jnp.tanh(gi[:, 2 * HID:] + r * gh[:, 2 * HID:])
    return (1.0 - z) * n + z * h


def _edge_prologue_body(uf_ref, weT_ref, be_ref, wihT_ref, bih_ref, bhh_ref,
                        out_ref):
    rel = jnp.dot(uf_ref[...], weT_ref[...],
                  preferred_element_type=jnp.float32) + be_ref[...]
    rel = jnp.maximum(rel, 0.0)
    gi = jnp.dot(rel, wihT_ref[...],
                 preferred_element_type=jnp.float32) + bih_ref[...]
    bhh = bhh_ref[...]
    r = _sigmoid(gi[:, :HID] + bhh[:, :HID])
    z = _sigmoid(gi[:, HID:2 * HID] + bhh[:, HID:2 * HID])
    n = jnp.tanh(gi[:, 2 * HID:] + r * bhh[:, 2 * HID:])
    out_ref[...] = (1.0 - z) * n


def _edge_prologue(uf, weT, be, wihT, bih, bhh):
    BE = 1024
    grid = (N_REL // BE,)
    return pl.pallas_call(
        _edge_prologue_body,
        grid=grid,
        in_specs=[
            pl.BlockSpec((BE, uf.shape[1]), lambda i: (i, 0)),
            pl.BlockSpec(weT.shape, lambda i: (0, 0)),
            pl.BlockSpec(be.shape, lambda i: (0, 0)),
            pl.BlockSpec(wihT.shape, lambda i: (0, 0)),
            pl.BlockSpec(bih.shape, lambda i: (0, 0)),
            pl.BlockSpec(bhh.shape, lambda i: (0, 0)),
        ],
        out_specs=pl.BlockSpec((BE, HID), lambda i: (i, 0)),
        out_shape=jax.ShapeDtypeStruct((N_REL, HID), jnp.float32),
    )(uf, weT, be, wihT, bih, bhh)


def _vert_prologue_body(x_ref, woT_ref, bo_ref, wihT_ref, bih_ref, bhh_ref,
                        out_ref):
    obj_rep = jnp.dot(x_ref[...], woT_ref[...],
                      preferred_element_type=jnp.float32) + bo_ref[...]
    gi = jnp.dot(obj_rep, wihT_ref[...],
                 preferred_element_type=jnp.float32) + bih_ref[...]
    bhh = bhh_ref[...]
    r = _sigmoid(gi[:, :HID] + bhh[:, :HID])
    z = _sigmoid(gi[:, HID:2 * HID] + bhh[:, HID:2 * HID])
    n = jnp.tanh(gi[:, 2 * HID:] + r * bhh[:, 2 * HID:])
    out_ref[...] = (1.0 - z) * n


def _vert_prologue(x, woT, bo, wihT, bih, bhh):
    return pl.pallas_call(
        _vert_prologue_body,
        out_shape=jax.ShapeDtypeStruct((N_OBJ, HID), jnp.float32),
    )(x, woT, bo, wihT, bih, bhh)


def _edge_iter_body(sub_ref, obj_ref, edge_ref, gv_ref, ge_ref, gb_ref,
                    wihT_ref, whhT_ref, bih_ref, bhh_ref,
                    eo_ref, po_ref, pi_ref):
    sub = sub_ref[...]
    obj = obj_ref[...]
    edge = edge_ref[...]
    gv = gv_ref[...]
    ge = ge_ref[...]
    gb = gb_ref[...]

    a_s = jnp.sum(sub * gv[0:1, :], axis=1, keepdims=True)
    b_o = jnp.sum(obj * gv[1:2, :], axis=1, keepdims=True)
    a_out = jnp.sum(sub * gv[2:3, :], axis=1, keepdims=True)
    b_in = jnp.sum(obj * gv[3:4, :], axis=1, keepdims=True)
    e_s = jnp.sum(edge * ge[0:1, :], axis=1, keepdims=True)
    e_o = jnp.sum(edge * ge[1:2, :], axis=1, keepdims=True)
    e_out = jnp.sum(edge * ge[2:3, :], axis=1, keepdims=True)
    e_in = jnp.sum(edge * ge[3:4, :], axis=1, keepdims=True)

    w_s = _sigmoid(a_s + e_s + gb[0:1, 0:1])
    w_o = _sigmoid(b_o + e_o + gb[0:1, 1:2])
    g_out = _sigmoid(a_out + e_out + gb[0:1, 2:3])
    g_in = _sigmoid(b_in + e_in + gb[0:1, 3:4])

    x_in = w_s * sub + w_o * obj
    gi = jnp.dot(x_in, wihT_ref[...],
                 preferred_element_type=jnp.float32) + bih_ref[...]
    gh = jnp.dot(edge, whhT_ref[...],
                 preferred_element_type=jnp.float32) + bhh_ref[...]
    eo_ref[...] = _gru_tail(gi, gh, edge)
    po_ref[...] = g_out * edge
    pi_ref[...] = g_in * edge


def _edge_iter(sub_v, obj_v, edge, gv, ge, gb, wihT, whhT, bih, bhh):
    BE = 2048
    grid = (N_REL // BE,)
    row = lambda i: (i, 0)
    rep = lambda i: (0, 0)
    return pl.pallas_call(
        _edge_iter_body,
        grid=grid,
        in_specs=[
            pl.BlockSpec((BE, HID), row),
            pl.BlockSpec((BE, HID), row),
            pl.BlockSpec((BE, HID), row),
            pl.BlockSpec(gv.shape, rep),
            pl.BlockSpec(ge.shape, rep),
            pl.BlockSpec(gb.shape, rep),
            pl.BlockSpec(wihT.shape, rep),
            pl.BlockSpec(whhT.shape, rep),
            pl.BlockSpec(bih.shape, rep),
            pl.BlockSpec(bhh.shape, rep),
        ],
        out_specs=[
            pl.BlockSpec((BE, HID), row),
            pl.BlockSpec((BE, HID), row),
            pl.BlockSpec((BE, HID), row),
        ],
        out_shape=[
            jax.ShapeDtypeStruct((N_REL, HID), jnp.float32),
            jax.ShapeDtypeStruct((N_REL, HID), jnp.float32),
            jax.ShapeDtypeStruct((N_REL, HID), jnp.float32),
        ],
    )(sub_v, obj_v, edge, gv, ge, gb, wihT, whhT, bih, bhh)


def _node_iter_body(p0_ref, p1_ref, vert_ref, wihT_ref, whhT_ref, bih_ref,
                    bhh_ref, out_ref):
    ctx = p0_ref[...] + p1_ref[...]
    vert = vert_ref[...]
    gi = jnp.dot(ctx, wihT_ref[...],
                 preferred_element_type=jnp.float32) + bih_ref[...]
    gh = jnp.dot(vert, whhT_ref[...],
                 preferred_element_type=jnp.float32) + bhh_ref[...]
    out_ref[...] = _gru_tail(gi, gh, vert)


def _node_iter(p0, p1, vert, wihT, whhT, bih, bhh):
    return pl.pallas_call(
        _node_iter_body,
        out_shape=jax.ShapeDtypeStruct((N_OBJ, HID), jnp.float32),
    )(p0, p1, vert, wihT, whhT, bih, bhh)


def _proj_body(x_ref, wT_ref, b_ref, out_ref):
    out_ref[...] = jnp.dot(x_ref[...], wT_ref[...],
                           preferred_element_type=jnp.float32) + b_ref[...]


def _proj_small(x, wT, b):
    return pl.pallas_call(
        _proj_body,
        out_shape=jax.ShapeDtypeStruct((x.shape[0], wT.shape[1]), jnp.float32),
    )(x, wT, b)


def _proj_edges(x, wT, b):
    BE = 4096
    grid = (N_REL // BE,)
    return pl.pallas_call(
        _proj_body,
        grid=grid,
        in_specs=[
            pl.BlockSpec((BE, HID), lambda i: (i, 0)),
            pl.BlockSpec(wT.shape, lambda i: (0, 0)),
            pl.BlockSpec(b.shape, lambda i: (0, 0)),
        ],
        out_specs=pl.BlockSpec((BE, wT.shape[1]), lambda i: (i, 0)),
        out_shape=jax.ShapeDtypeStruct((N_REL, wT.shape[1]), jnp.float32),
    )(x, wT, b)


_SC_MESH = plsc.VectorSubcoreMesh(core_axis_name="c", subcore_axis_name="s")


@functools.partial(
    pl.kernel,
    mesh=_SC_MESH,
    out_type=[
        jax.ShapeDtypeStruct((N_REL, HID), jnp.float32),
        jax.ShapeDtypeStruct((N_REL, HID), jnp.float32),
    ],
    scratch_types=[
        pltpu.VMEM((EPW,), jnp.int32),
        pltpu.VMEM((EPW, HID), jnp.float32),
        pltpu.SemaphoreType.DMA,
    ],
)
def _sc_gather(table_hbm, sidx_hbm, oidx_hbm, sub_out, obj_out,
               idx_v, rows_v, sem):
    wid = lax.axis_index("s") * NC + lax.axis_index("c")
    base = wid * EPW
    pltpu.sync_copy(sidx_hbm.at[pl.ds(base, EPW)], idx_v)
    pltpu.async_copy(table_hbm.at[idx_v], rows_v, sem).wait()
    pltpu.sync_copy(rows_v, sub_out.at[pl.ds(base, EPW)])
    pltpu.sync_copy(oidx_hbm.at[pl.ds(base, EPW)], idx_v)
    pltpu.async_copy(table_hbm.at[idx_v], rows_v, sem).wait()
    pltpu.sync_copy(rows_v, obj_out.at[pl.ds(base, EPW)])


_ROWS_PER_TILE = N_OBJ // NS


@functools.partial(
    pl.kernel,
    mesh=_SC_MESH,
    out_type=jax.ShapeDtypeStruct((NC * N_OBJ, HID), jnp.float32),
    scratch_types=[
        pltpu.VMEM((EPW,), jnp.int32),
        pltpu.VMEM((EPW, HID), jnp.float32),
        pltpu.VMEM_SHARED((N_OBJ, HID), jnp.float32),
    ],
)
def _sc_scatter(po_hbm, pi_hbm, sidx_hbm, oidx_hbm, zeros_hbm, out_hbm,
                idx_v, rows_v, acc):
    c = lax.axis_index("c")
    s = lax.axis_index("s")
    pltpu.sync_copy(zeros_hbm, acc.at[pl.ds(s * _ROWS_PER_TILE,
                                            _ROWS_PER_TILE)])
    plsc.subcore_barrier()
    base = c * (N_REL // NC) + s * EPW
    pltpu.sync_copy(sidx_hbm.at[pl.ds(base, EPW)], idx_v)
    pltpu.sync_copy(po_hbm.at[pl.ds(base, EPW)], rows_v)
    pltpu.sync_copy(rows_v, acc.at[idx_v], add=True)
    pltpu.sync_copy(oidx_hbm.at[pl.ds(base, EPW)], idx_v)
    pltpu.sync_copy(pi_hbm.at[pl.ds(base, EPW)], rows_v)
    pltpu.sync_copy(rows_v, acc.at[idx_v], add=True)
    plsc.subcore_barrier()
    src = acc.at[pl.ds(s * _ROWS_PER_TILE, _ROWS_PER_TILE)]
    pltpu.sync_copy(src, out_hbm.at[pl.ds(c * N_OBJ + s * _ROWS_PER_TILE,
                                          _ROWS_PER_TILE)])


def kernel(x, union_features, rel_pair_idxs, obj_unary_w, obj_unary_b,
           edge_unary_w, edge_unary_b, node_w_ih, node_w_hh, node_b_ih,
           node_b_hh, edge_w_ih, edge_w_hh, edge_b_ih, edge_b_hh, sub_w,
           sub_b, obj_w, obj_b, out_w, out_b, in_w, in_b, obj_fc_w,
           obj_fc_b, rel_fc_w, rel_fc_b):
    f32 = jnp.float32
    sidx = rel_pair_idxs[:, 0].astype(jnp.int32)
    oidx = rel_pair_idxs[:, 1].astype(jnp.int32)

    weT = edge_unary_w.T
    woT = obj_unary_w.T
    e_wihT = edge_w_ih.T
    e_whhT = edge_w_hh.T
    n_wihT = node_w_ih.T
    n_whhT = node_w_hh.T
    e_bih = edge_b_ih.reshape(1, -1)
    e_bhh = edge_b_hh.reshape(1, -1)
    n_bih = node_b_ih.reshape(1, -1)
    n_bhh = node_b_hh.reshape(1, -1)
    be = edge_unary_b.reshape(1, -1)
    bo = obj_unary_b.reshape(1, -1)

    gv = jnp.zeros((8, HID), f32).at[:4].set(
        jnp.stack([sub_w[0, :HID], obj_w[0, :HID],
                   out_w[0, :HID], in_w[0, :HID]]))
    ge = jnp.zeros((8, HID), f32).at[:4].set(
        jnp.stack([sub_w[0, HID:], obj_w[0, HID:],
                   out_w[0, HID:], in_w[0, HID:]]))
    gb = jnp.zeros((1, HID), f32).at[0, :4].set(
        jnp.stack([sub_b[0], obj_b[0], out_b[0], in_b[0]]))

    edge = _edge_prologue(union_features, weT, be, e_wihT, e_bih, e_bhh)
    vert = _vert_prologue(x, woT, bo, n_wihT, n_bih, n_bhh)

    zeros_tile = jnp.zeros((_ROWS_PER_TILE, HID), f32)

    for _ in range(3):
        sub_v, obj_v = _sc_gather(vert, sidx, oidx)
        edge_new, pre_out, pre_in = _edge_iter(
            sub_v, obj_v, edge, gv, ge, gb, e_wihT, e_whhT, e_bih, e_bhh)
        parts = _sc_scatter(pre_out, pre_in, sidx, oidx, zeros_tile)
        vert = _node_iter(parts[:N_OBJ], parts[N_OBJ:], vert,
                          n_wihT, n_whhT, n_bih, n_bhh)
        edge = edge_new

    obj_fcT = jnp.zeros((HID, 256), f32).at[:, :NUM_OBJ_CLS].set(obj_fc_w.T)
    obj_fcb = jnp.zeros((1, 256), f32).at[0, :NUM_OBJ_CLS].set(obj_fc_b)
    rel_fcT = jnp.zeros((HID, 128), f32).at[:, :NUM_REL_CLS].set(rel_fc_w.T)
    rel_fcb = jnp.zeros((1, 128), f32).at[0, :NUM_REL_CLS].set(rel_fc_b)

    obj_dists = _proj_small(vert, obj_fcT, obj_fcb)[:, :NUM_OBJ_CLS]
    rel_dists = _proj_edges(edge, rel_fcT, rel_fcb)[:, :NUM_REL_CLS]
    return (obj_dists, rel_dists)


# speedup vs baseline: 3.5417x; 1.0227x over previous
import functools

import jax
import jax.numpy as jnp
from jax import lax
from jax.experimental import pallas as pl
from jax.experimental.pallas import tpu as pltpu
from jax.experimental.pallas import tpu_sc as plsc

HID = 128
N_OBJ = 2048
N_REL = 16384
NUM_OBJ_CLS = 151
NUM_REL_CLS = 51

NC = 2
NS = 16
NW = NC * NS
EPW = N_REL // NW


def _sigmoid(x):
    return jax.nn.sigmoid(x)


def _gru_tail(gi, gh, h):
    r = _sigmoid(gi[:, :HID] + gh[:, :HID])
    z = _sigmoid(gi[:, HID:2 * HID] + gh[:, HID:2 * HID])
    n = jnp.tanh(gi[:, 2 * HID:] + r * gh[:, 2 * HID:])
    return (1.0 - z) * n + z * h


def _edge_prologue_body(uf_ref, weT_ref, be_ref, wihT_ref, bih_ref, bhh_ref,
                        out_ref):
    rel = jnp.dot(uf_ref[...], weT_ref[...],
                  preferred_element_type=jnp.float32) + be_ref[...]
    rel = jnp.maximum(rel, 0.0)
    gi = jnp.dot(rel, wihT_ref[...],
                 preferred_element_type=jnp.float32) + bih_ref[...]
    bhh = bhh_ref[...]
    r = _sigmoid(gi[:, :HID] + bhh[:, :HID])
    z = _sigmoid(gi[:, HID:2 * HID] + bhh[:, HID:2 * HID])
    n = jnp.tanh(gi[:, 2 * HID:] + r * bhh[:, 2 * HID:])
    out_ref[...] = (1.0 - z) * n


def _edge_prologue(uf, weT, be, wihT, bih, bhh):
    BE = 1024
    grid = (N_REL // BE,)
    return pl.pallas_call(
        _edge_prologue_body,
        grid=grid,
        in_specs=[
            pl.BlockSpec((BE, uf.shape[1]), lambda i: (i, 0)),
            pl.BlockSpec(weT.shape, lambda i: (0, 0)),
            pl.BlockSpec(be.shape, lambda i: (0, 0)),
            pl.BlockSpec(wihT.shape, lambda i: (0, 0)),
            pl.BlockSpec(bih.shape, lambda i: (0, 0)),
            pl.BlockSpec(bhh.shape, lambda i: (0, 0)),
        ],
        out_specs=pl.BlockSpec((BE, HID), lambda i: (i, 0)),
        out_shape=jax.ShapeDtypeStruct((N_REL, HID), jnp.float32),
    )(uf, weT, be, wihT, bih, bhh)


def _vert_prologue_body(x_ref, woT_ref, bo_ref, wihT_ref, bih_ref, bhh_ref,
                        out_ref):
    obj_rep = jnp.dot(x_ref[...], woT_ref[...],
                      preferred_element_type=jnp.float32) + bo_ref[...]
    gi = jnp.dot(obj_rep, wihT_ref[...],
                 preferred_element_type=jnp.float32) + bih_ref[...]
    bhh = bhh_ref[...]
    r = _sigmoid(gi[:, :HID] + bhh[:, :HID])
    z = _sigmoid(gi[:, HID:2 * HID] + bhh[:, HID:2 * HID])
    n = jnp.tanh(gi[:, 2 * HID:] + r * bhh[:, 2 * HID:])
    out_ref[...] = (1.0 - z) * n


def _vert_prologue(x, woT, bo, wihT, bih, bhh):
    return pl.pallas_call(
        _vert_prologue_body,
        out_shape=jax.ShapeDtypeStruct((N_OBJ, HID), jnp.float32),
    )(x, woT, bo, wihT, bih, bhh)


def _edge_iter_body(sub_ref, obj_ref, edge_ref, gv_ref, ge_ref, gb_ref,
                    wihT_ref, whhT_ref, bih_ref, bhh_ref,
                    eo_ref, po_ref, pi_ref):
    sub = sub_ref[...]
    obj = obj_ref[...]
    edge = edge_ref[...]
    gv = gv_ref[...]
    ge = ge_ref[...]
    gb = gb_ref[...]

    a_s = jnp.sum(sub * gv[0:1, :], axis=1, keepdims=True)
    b_o = jnp.sum(obj * gv[1:2, :], axis=1, keepdims=True)
    a_out = jnp.sum(sub * gv[2:3, :], axis=1, keepdims=True)
    b_in = jnp.sum(obj * gv[3:4, :], axis=1, keepdims=True)
    e_s = jnp.sum(edge * ge[0:1, :], axis=1, keepdims=True)
    e_o = jnp.sum(edge * ge[1:2, :], axis=1, keepdims=True)
    e_out = jnp.sum(edge * ge[2:3, :], axis=1, keepdims=True)
    e_in = jnp.sum(edge * ge[3:4, :], axis=1, keepdims=True)

    w_s = _sigmoid(a_s + e_s + gb[0:1, 0:1])
    w_o = _sigmoid(b_o + e_o + gb[0:1, 1:2])
    g_out = _sigmoid(a_out + e_out + gb[0:1, 2:3])
    g_in = _sigmoid(b_in + e_in + gb[0:1, 3:4])

    x_in = w_s * sub + w_o * obj
    gi = jnp.dot(x_in, wihT_ref[...],
                 preferred_element_type=jnp.float32) + bih_ref[...]
    gh = jnp.dot(edge, whhT_ref[...],
                 preferred_element_type=jnp.float32) + bhh_ref[...]
    eo_ref[...] = _gru_tail(gi, gh, edge)
    po_ref[...] = g_out * edge
    pi_ref[...] = g_in * edge


def _edge_iter(sub_v, obj_v, edge, gv, ge, gb, wihT, whhT, bih, bhh):
    BE = 2048
    grid = (N_REL // BE,)
    row = lambda i: (i, 0)
    rep = lambda i: (0, 0)
    return pl.pallas_call(
        _edge_iter_body,
        grid=grid,
        in_specs=[
            pl.BlockSpec((BE, HID), row),
            pl.BlockSpec((BE, HID), row),
            pl.BlockSpec((BE, HID), row),
            pl.BlockSpec(gv.shape, rep),
            pl.BlockSpec(ge.shape, rep),
            pl.BlockSpec(gb.shape, rep),
            pl.BlockSpec(wihT.shape, rep),
            pl.BlockSpec(whhT.shape, rep),
            pl.BlockSpec(bih.shape, rep),
            pl.BlockSpec(bhh.shape, rep),
        ],
        out_specs=[
            pl.BlockSpec((BE, HID), row),
            pl.BlockSpec((BE, HID), row),
            pl.BlockSpec((BE, HID), row),
        ],
        out_shape=[
            jax.ShapeDtypeStruct((N_REL, HID), jnp.float32),
            jax.ShapeDtypeStruct((N_REL, HID), jnp.float32),
            jax.ShapeDtypeStruct((N_REL, HID), jnp.float32),
        ],
    )(sub_v, obj_v, edge, gv, ge, gb, wihT, whhT, bih, bhh)


def _edge_iter_last_body(sub_ref, obj_ref, edge_ref, gv_ref, ge_ref, gb_ref,
                         wihT_ref, whhT_ref, bih_ref, bhh_ref,
                         fcT_ref, fcb_ref, rd_ref, po_ref, pi_ref):
    sub = sub_ref[...]
    obj = obj_ref[...]
    edge = edge_ref[...]
    gv = gv_ref[...]
    ge = ge_ref[...]
    gb = gb_ref[...]

    a_s = jnp.sum(sub * gv[0:1, :], axis=1, keepdims=True)
    b_o = jnp.sum(obj * gv[1:2, :], axis=1, keepdims=True)
    a_out = jnp.sum(sub * gv[2:3, :], axis=1, keepdims=True)
    b_in = jnp.sum(obj * gv[3:4, :], axis=1, keepdims=True)
    e_s = jnp.sum(edge * ge[0:1, :], axis=1, keepdims=True)
    e_o = jnp.sum(edge * ge[1:2, :], axis=1, keepdims=True)
    e_out = jnp.sum(edge * ge[2:3, :], axis=1, keepdims=True)
    e_in = jnp.sum(edge * ge[3:4, :], axis=1, keepdims=True)

    w_s = _sigmoid(a_s + e_s + gb[0:1, 0:1])
    w_o = _sigmoid(b_o + e_o + gb[0:1, 1:2])
    g_out = _sigmoid(a_out + e_out + gb[0:1, 2:3])
    g_in = _sigmoid(b_in + e_in + gb[0:1, 3:4])

    x_in = w_s * sub + w_o * obj
    gi = jnp.dot(x_in, wihT_ref[...],
                 preferred_element_type=jnp.float32) + bih_ref[...]
    gh = jnp.dot(edge, whhT_ref[...],
                 preferred_element_type=jnp.float32) + bhh_ref[...]
    edge_new = _gru_tail(gi, gh, edge)
    rd_ref[...] = jnp.dot(edge_new, fcT_ref[...],
                          preferred_element_type=jnp.float32) + fcb_ref[...]
    po_ref[...] = g_out * edge
    pi_ref[...] = g_in * edge


def _edge_iter_last(sub_v, obj_v, edge, gv, ge, gb, wihT, whhT, bih, bhh,
                    fcT, fcb):
    BE = 2048
    grid = (N_REL // BE,)
    row = lambda i: (i, 0)
    rep = lambda i: (0, 0)
    return pl.pallas_call(
        _edge_iter_last_body,
        grid=grid,
        in_specs=[
            pl.BlockSpec((BE, HID), row),
            pl.BlockSpec((BE, HID), row),
            pl.BlockSpec((BE, HID), row),
            pl.BlockSpec(gv.shape, rep),
            pl.BlockSpec(ge.shape, rep),
            pl.BlockSpec(gb.shape, rep),
            pl.BlockSpec(wihT.shape, rep),
            pl.BlockSpec(whhT.shape, rep),
            pl.BlockSpec(bih.shape, rep),
            pl.BlockSpec(bhh.shape, rep),
            pl.BlockSpec(fcT.shape, rep),
            pl.BlockSpec(fcb.shape, rep),
        ],
        out_specs=[
            pl.BlockSpec((BE, fcT.shape[1]), row),
            pl.BlockSpec((BE, HID), row),
            pl.BlockSpec((BE, HID), row),
        ],
        out_shape=[
            jax.ShapeDtypeStruct((N_REL, fcT.shape[1]), jnp.float32),
            jax.ShapeDtypeStruct((N_REL, HID), jnp.float32),
            jax.ShapeDtypeStruct((N_REL, HID), jnp.float32),
        ],
    )(sub_v, obj_v, edge, gv, ge, gb, wihT, whhT, bih, bhh, fcT, fcb)


def _node_iter_body(p0_ref, p1_ref, vert_ref, wihT_ref, whhT_ref, bih_ref,
                    bhh_ref, out_ref):
    ctx = p0_ref[...] + p1_ref[...]
    vert = vert_ref[...]
    gi = jnp.dot(ctx, wihT_ref[...],
                 preferred_element_type=jnp.float32) + bih_ref[...]
    gh = jnp.dot(vert, whhT_ref[...],
                 preferred_element_type=jnp.float32) + bhh_ref[...]
    out_ref[...] = _gru_tail(gi, gh, vert)


def _node_iter(p0, p1, vert, wihT, whhT, bih, bhh):
    return pl.pallas_call(
        _node_iter_body,
        out_shape=jax.ShapeDtypeStruct((N_OBJ, HID), jnp.float32),
    )(p0, p1, vert, wihT, whhT, bih, bhh)


def _node_iter_last_body(p0_ref, p1_ref, vert_ref, wihT_ref, whhT_ref,
                         bih_ref, bhh_ref, fcT_ref, fcb_ref, out_ref):
    ctx = p0_ref[...] + p1_ref[...]
    vert = vert_ref[...]
    gi = jnp.dot(ctx, wihT_ref[...],
                 preferred_element_type=jnp.float32) + bih_ref[...]
    gh = jnp.dot(vert, whhT_ref[...],
                 preferred_element_type=jnp.float32) + bhh_ref[...]
    vert_new = _gru_tail(gi, gh, vert)
    out_ref[...] = jnp.dot(vert_new, fcT_ref[...],
                           preferred_element_type=jnp.float32) + fcb_ref[...]


def _node_iter_last(p0, p1, vert, wihT, whhT, bih, bhh, fcT, fcb):
    return pl.pallas_call(
        _node_iter_last_body,
        out_shape=jax.ShapeDtypeStruct((N_OBJ, fcT.shape[1]), jnp.float32),
    )(p0, p1, vert, wihT, whhT, bih, bhh, fcT, fcb)


def _proj_body(x_ref, wT_ref, b_ref, out_ref):
    out_ref[...] = jnp.dot(x_ref[...], wT_ref[...],
                           preferred_element_type=jnp.float32) + b_ref[...]


def _proj_small(x, wT, b):
    return pl.pallas_call(
        _proj_body,
        out_shape=jax.ShapeDtypeStruct((x.shape[0], wT.shape[1]), jnp.float32),
    )(x, wT, b)


def _proj_edges(x, wT, b):
    BE = 4096
    grid = (N_REL // BE,)
    return pl.pallas_call(
        _proj_body,
        grid=grid,
        in_specs=[
            pl.BlockSpec((BE, HID), lambda i: (i, 0)),
            pl.BlockSpec(wT.shape, lambda i: (0, 0)),
            pl.BlockSpec(b.shape, lambda i: (0, 0)),
        ],
        out_specs=pl.BlockSpec((BE, wT.shape[1]), lambda i: (i, 0)),
        out_shape=jax.ShapeDtypeStruct((N_REL, wT.shape[1]), jnp.float32),
    )(x, wT, b)


_SC_MESH = plsc.VectorSubcoreMesh(core_axis_name="c", subcore_axis_name="s")
_CH = 256


@functools.partial(
    pl.kernel,
    mesh=_SC_MESH,
    out_type=[
        jax.ShapeDtypeStruct((N_REL, HID), jnp.float32),
        jax.ShapeDtypeStruct((N_REL, HID), jnp.float32),
    ],
    scratch_types=[
        pltpu.VMEM((_CH,), jnp.int32),
        pltpu.VMEM((_CH,), jnp.int32),
        pltpu.VMEM((_CH,), jnp.int32),
        pltpu.VMEM((_CH,), jnp.int32),
        pltpu.VMEM((_CH, HID), jnp.float32),
        pltpu.VMEM((_CH, HID), jnp.float32),
        pltpu.SemaphoreType.DMA,
        pltpu.SemaphoreType.DMA,
        pltpu.SemaphoreType.DMA,
        pltpu.SemaphoreType.DMA,
    ],
)
def _sc_gather(table_hbm, sidx_hbm, oidx_hbm, sub_out, obj_out,
               i0, i1, i2, i3, buf_a, buf_b, gs_a, gs_b, ws_a, ws_b):
    wid = lax.axis_index("s") * NC + lax.axis_index("c")
    base = wid * EPW
    pltpu.sync_copy(sidx_hbm.at[pl.ds(base, _CH)], i0)
    g0 = pltpu.async_copy(table_hbm.at[i0], buf_a, gs_a)
    pltpu.sync_copy(sidx_hbm.at[pl.ds(base + _CH, _CH)], i1)
    g1 = pltpu.async_copy(table_hbm.at[i1], buf_b, gs_b)
    pltpu.sync_copy(oidx_hbm.at[pl.ds(base, _CH)], i2)
    pltpu.sync_copy(oidx_hbm.at[pl.ds(base + _CH, _CH)], i3)
    g0.wait()
    w0 = pltpu.async_copy(buf_a, sub_out.at[pl.ds(base, _CH)], ws_a)
    g1.wait()
    w1 = pltpu.async_copy(buf_b, sub_out.at[pl.ds(base + _CH, _CH)], ws_b)
    w0.wait()
    g2 = pltpu.async_copy(table_hbm.at[i2], buf_a, gs_a)
    w1.wait()
    g3 = pltpu.async_copy(table_hbm.at[i3], buf_b, gs_b)
    g2.wait()
    w2 = pltpu.async_copy(buf_a, obj_out.at[pl.ds(base, _CH)], ws_a)
    g3.wait()
    w3 = pltpu.async_copy(buf_b, obj_out.at[pl.ds(base + _CH, _CH)], ws_b)
    w2.wait()
    w3.wait()


_ROWS_PER_TILE = N_OBJ // NS


@functools.partial(
    pl.kernel,
    mesh=_SC_MESH,
    out_type=jax.ShapeDtypeStruct((NC * N_OBJ, HID), jnp.float32),
    scratch_types=[
        pltpu.VMEM((_CH,), jnp.int32),
        pltpu.VMEM((_CH,), jnp.int32),
        pltpu.VMEM((_CH,), jnp.int32),
        pltpu.VMEM((_CH,), jnp.int32),
        pltpu.VMEM((_CH, HID), jnp.float32),
        pltpu.VMEM((_CH, HID), jnp.float32),
        pltpu.VMEM_SHARED((N_OBJ, HID), jnp.float32),
        pltpu.SemaphoreType.DMA,
        pltpu.SemaphoreType.DMA,
        pltpu.SemaphoreType.DMA,
    ],
)
def _sc_scatter(po_hbm, pi_hbm, sidx_hbm, oidx_hbm, zeros_hbm, out_hbm,
                i_s0, i_s1, i_o0, i_o1, buf_a, buf_b, acc, rs_a, rs_b, zs):
    c = lax.axis_index("c")
    s = lax.axis_index("s")
    z = pltpu.async_copy(zeros_hbm, acc.at[pl.ds(s * _ROWS_PER_TILE,
                                                 _ROWS_PER_TILE)], zs)
    base = c * (N_REL // NC) + s * EPW
    pltpu.sync_copy(sidx_hbm.at[pl.ds(base, _CH)], i_s0)
    pltpu.sync_copy(sidx_hbm.at[pl.ds(base + _CH, _CH)], i_s1)
    pltpu.sync_copy(oidx_hbm.at[pl.ds(base, _CH)], i_o0)
    pltpu.sync_copy(oidx_hbm.at[pl.ds(base + _CH, _CH)], i_o1)
    r0 = pltpu.async_copy(po_hbm.at[pl.ds(base, _CH)], buf_a, rs_a)
    r1 = pltpu.async_copy(po_hbm.at[pl.ds(base + _CH, _CH)], buf_b, rs_b)
    z.wait()
    plsc.subcore_barrier()
    r0.wait()
    pltpu.sync_copy(buf_a, acc.at[i_s0], add=True)
    r2 = pltpu.async_copy(pi_hbm.at[pl.ds(base, _CH)], buf_a, rs_a)
    r1.wait()
    pltpu.sync_copy(buf_b, acc.at[i_s1], add=True)
    r3 = pltpu.async_copy(pi_hbm.at[pl.ds(base + _CH, _CH)], buf_b, rs_b)
    r2.wait()
    pltpu.sync_copy(buf_a, acc.at[i_o0], add=True)
    r3.wait()
    pltpu.sync_copy(buf_b, acc.at[i_o1], add=True)
    plsc.subcore_barrier()
    src = acc.at[pl.ds(s * _ROWS_PER_TILE, _ROWS_PER_TILE)]
    pltpu.sync_copy(src, out_hbm.at[pl.ds(c * N_OBJ + s * _ROWS_PER_TILE,
                                          _ROWS_PER_TILE)])


def kernel(x, union_features, rel_pair_idxs, obj_unary_w, obj_unary_b,
           edge_unary_w, edge_unary_b, node_w_ih, node_w_hh, node_b_ih,
           node_b_hh, edge_w_ih, edge_w_hh, edge_b_ih, edge_b_hh, sub_w,
           sub_b, obj_w, obj_b, out_w, out_b, in_w, in_b, obj_fc_w,
           obj_fc_b, rel_fc_w, rel_fc_b):
    f32 = jnp.float32
    sidx = rel_pair_idxs[:, 0].astype(jnp.int32)
    oidx = rel_pair_idxs[:, 1].astype(jnp.int32)

    weT = edge_unary_w.T
    woT = obj_unary_w.T
    e_wihT = edge_w_ih.T
    e_whhT = edge_w_hh.T
    n_wihT = node_w_ih.T
    n_whhT = node_w_hh.T
    e_bih = edge_b_ih.reshape(1, -1)
    e_bhh = edge_b_hh.reshape(1, -1)
    n_bih = node_b_ih.reshape(1, -1)
    n_bhh = node_b_hh.reshape(1, -1)
    be = edge_unary_b.reshape(1, -1)
    bo = obj_unary_b.reshape(1, -1)

    gv = jnp.zeros((8, HID), f32).at[:4].set(
        jnp.stack([sub_w[0, :HID], obj_w[0, :HID],
                   out_w[0, :HID], in_w[0, :HID]]))
    ge = jnp.zeros((8, HID), f32).at[:4].set(
        jnp.stack([sub_w[0, HID:], obj_w[0, HID:],
                   out_w[0, HID:], in_w[0, HID:]]))
    gb = jnp.zeros((1, HID), f32).at[0, :4].set(
        jnp.stack([sub_b[0], obj_b[0], out_b[0], in_b[0]]))

    obj_fcT = jnp.zeros((HID, 256), f32).at[:, :NUM_OBJ_CLS].set(obj_fc_w.T)
    obj_fcb = jnp.zeros((1, 256), f32).at[0, :NUM_OBJ_CLS].set(obj_fc_b)
    rel_fcT = jnp.zeros((HID, 128), f32).at[:, :NUM_REL_CLS].set(rel_fc_w.T)
    rel_fcb = jnp.zeros((1, 128), f32).at[0, :NUM_REL_CLS].set(rel_fc_b)

    edge = _edge_prologue(union_features, weT, be, e_wihT, e_bih, e_bhh)
    vert = _vert_prologue(x, woT, bo, n_wihT, n_bih, n_bhh)

    zeros_tile = jnp.zeros((_ROWS_PER_TILE, HID), f32)

    for _ in range(2):
        sub_v, obj_v = _sc_gather(vert, sidx, oidx)
        edge_new, pre_out, pre_in = _edge_iter(
            sub_v, obj_v, edge, gv, ge, gb, e_wihT, e_whhT, e_bih, e_bhh)
        parts = _sc_scatter(pre_out, pre_in, sidx, oidx, zeros_tile)
        vert = _node_iter(parts[:N_OBJ], parts[N_OBJ:], vert,
                          n_wihT, n_whhT, n_bih, n_bhh)
        edge = edge_new

    sub_v, obj_v = _sc_gather(vert, sidx, oidx)
    rel_pad, pre_out, pre_in = _edge_iter_last(
        sub_v, obj_v, edge, gv, ge, gb, e_wihT, e_whhT, e_bih, e_bhh,
        rel_fcT, rel_fcb)
    parts = _sc_scatter(pre_out, pre_in, sidx, oidx, zeros_tile)
    obj_pad = _node_iter_last(parts[:N_OBJ], parts[N_OBJ:], vert,
                              n_wihT, n_whhT, n_bih, n_bhh, obj_fcT, obj_fcb)

    return (obj_pad[:, :NUM_OBJ_CLS], rel_pad[:, :NUM_REL_CLS])


# speedup vs baseline: 4.0873x; 1.1541x over previous
import functools

import jax
import jax.numpy as jnp
from jax import lax
from jax.experimental import pallas as pl
from jax.experimental.pallas import tpu as pltpu
from jax.experimental.pallas import tpu_sc as plsc

HID = 128
N_OBJ = 2048
N_REL = 16384
HALF = N_REL // 2
NUM_OBJ_CLS = 151
NUM_REL_CLS = 51

NC = 2
NS = 16
NW = NC * NS
EPW = HALF // NW
_CH = EPW // 2
_ROWS_PER_TILE = N_OBJ // NS

_DN_T = (((1,), (1,)), ((), ()))


def _sigmoid(x):
    return jax.nn.sigmoid(x)


def _gru_tail(gi, gh, h):
    r = _sigmoid(gi[:, :HID] + gh[:, :HID])
    z = _sigmoid(gi[:, HID:2 * HID] + gh[:, HID:2 * HID])
    n = jnp.tanh(gi[:, 2 * HID:] + r * gh[:, 2 * HID:])
    return (1.0 - z) * n + z * h


def _dotT(x, w):
    return lax.dot_general(x, w, _DN_T, preferred_element_type=jnp.float32)


def _edge_prologue_body(uf_ref, we_ref, be_ref, wih_ref, bih_ref, bhh_ref,
                        out_ref):
    rel = jnp.maximum(_dotT(uf_ref[...], we_ref[...]) + be_ref[...], 0.0)
    gi = _dotT(rel, wih_ref[...]) + bih_ref[...]
    bhh = bhh_ref[...]
    r = _sigmoid(gi[:, :HID] + bhh[:, :HID])
    z = _sigmoid(gi[:, HID:2 * HID] + bhh[:, HID:2 * HID])
    n = jnp.tanh(gi[:, 2 * HID:] + r * bhh[:, 2 * HID:])
    out_ref[...] = (1.0 - z) * n


def _edge_prologue(uf, we, be, wih, bih, bhh, half):
    BE = 1024
    nblk = HALF // BE
    off = half * nblk
    return pl.pallas_call(
        _edge_prologue_body,
        grid=(nblk,),
        in_specs=[
            pl.BlockSpec((BE, uf.shape[1]), lambda i: (i + off, 0)),
            pl.BlockSpec(we.shape, lambda i: (0, 0)),
            pl.BlockSpec(be.shape, lambda i: (0, 0)),
            pl.BlockSpec(wih.shape, lambda i: (0, 0)),
            pl.BlockSpec(bih.shape, lambda i: (0, 0)),
            pl.BlockSpec(bhh.shape, lambda i: (0, 0)),
        ],
        out_specs=pl.BlockSpec((BE, HID), lambda i: (i, 0)),
        out_shape=jax.ShapeDtypeStruct((HALF, HID), jnp.float32),
    )(uf, we, be, wih, bih, bhh)


def _vert_prologue_body(x_ref, wo_ref, bo_ref, wih_ref, bih_ref, bhh_ref,
                        out_ref):
    obj_rep = _dotT(x_ref[...], wo_ref[...]) + bo_ref[...]
    gi = _dotT(obj_rep, wih_ref[...]) + bih_ref[...]
    bhh = bhh_ref[...]
    r = _sigmoid(gi[:, :HID] + bhh[:, :HID])
    z = _sigmoid(gi[:, HID:2 * HID] + bhh[:, HID:2 * HID])
    n = jnp.tanh(gi[:, 2 * HID:] + r * bhh[:, 2 * HID:])
    out_ref[...] = (1.0 - z) * n


def _vert_prologue(x, wo, bo, wih, bih, bhh):
    return pl.pallas_call(
        _vert_prologue_body,
        out_shape=jax.ShapeDtypeStruct((N_OBJ, HID), jnp.float32),
    )(x, wo, bo, wih, bih, bhh)


def _edge_gates(sub, obj, edge, gvT, geT, gb):
    su = jnp.dot(sub, gvT, preferred_element_type=jnp.float32)
    ob = jnp.dot(obj, gvT, preferred_element_type=jnp.float32)
    ed = jnp.dot(edge, geT, preferred_element_type=jnp.float32)
    w_s = _sigmoid(su[:, 0:1] + ed[:, 0:1] + gb[0:1, 0:1])
    w_o = _sigmoid(ob[:, 1:2] + ed[:, 1:2] + gb[0:1, 1:2])
    g_out = _sigmoid(su[:, 2:3] + ed[:, 2:3] + gb[0:1, 2:3])
    g_in = _sigmoid(ob[:, 3:4] + ed[:, 3:4] + gb[0:1, 3:4])
    return w_s, w_o, g_out, g_in


def _edge_iter_body(sub_ref, obj_ref, edge_ref, gvT_ref, geT_ref, gb_ref,
                    wih_ref, whh_ref, bih_ref, bhh_ref,
                    eo_ref, po_ref, pi_ref):
    sub = sub_ref[...]
    obj = obj_ref[...]
    edge = edge_ref[...]
    w_s, w_o, g_out, g_in = _edge_gates(sub, obj, edge, gvT_ref[...],
                                        geT_ref[...], gb_ref[...])
    x_in = w_s * sub + w_o * obj
    gi = _dotT(x_in, wih_ref[...]) + bih_ref[...]
    gh = _dotT(edge, whh_ref[...]) + bhh_ref[...]
    eo_ref[...] = _gru_tail(gi, gh, edge)
    po_ref[...] = g_out * edge
    pi_ref[...] = g_in * edge


def _edge_iter(sub_v, obj_v, edge, gvT, geT, gb, wih, whh, bih, bhh):
    BE = 2048
    grid = (HALF // BE,)
    row = lambda i: (i, 0)
    rep = lambda i: (0, 0)
    return pl.pallas_call(
        _edge_iter_body,
        grid=grid,
        in_specs=[
            pl.BlockSpec((BE, HID), row),
            pl.BlockSpec((BE, HID), row),
            pl.BlockSpec((BE, HID), row),
            pl.BlockSpec(gvT.shape, rep),
            pl.BlockSpec(geT.shape, rep),
            pl.BlockSpec(gb.shape, rep),
            pl.BlockSpec(wih.shape, rep),
            pl.BlockSpec(whh.shape, rep),
            pl.BlockSpec(bih.shape, rep),
            pl.BlockSpec(bhh.shape, rep),
        ],
        out_specs=[
            pl.BlockSpec((BE, HID), row),
            pl.BlockSpec((BE, HID), row),
            pl.BlockSpec((BE, HID), row),
        ],
        out_shape=[
            jax.ShapeDtypeStruct((HALF, HID), jnp.float32),
            jax.ShapeDtypeStruct((HALF, HID), jnp.float32),
            jax.ShapeDtypeStruct((HALF, HID), jnp.float32),
        ],
    )(sub_v, obj_v, edge, gvT, geT, gb, wih, whh, bih, bhh)


def _edge_iter_last_body(sub_ref, obj_ref, edge_ref, gvT_ref, geT_ref, gb_ref,
                         wih_ref, whh_ref, bih_ref, bhh_ref,
                         fc_ref, fcb_ref, rd_ref, po_ref, pi_ref):
    sub = sub_ref[...]
    obj = obj_ref[...]
    edge = edge_ref[...]
    w_s, w_o, g_out, g_in = _edge_gates(sub, obj, edge, gvT_ref[...],
                                        geT_ref[...], gb_ref[...])
    x_in = w_s * sub + w_o * obj
    gi = _dotT(x_in, wih_ref[...]) + bih_ref[...]
    gh = _dotT(edge, whh_ref[...]) + bhh_ref[...]
    edge_new = _gru_tail(gi, gh, edge)
    rd_ref[...] = _dotT(edge_new, fc_ref[...]) + fcb_ref[...]
    po_ref[...] = g_out * edge
    pi_ref[...] = g_in * edge


def _edge_iter_last(sub_v, obj_v, edge, gvT, geT, gb, wih, whh, bih, bhh,
                    fc, fcb):
    BE = 2048
    grid = (HALF // BE,)
    row = lambda i: (i, 0)
    rep = lambda i: (0, 0)
    ncls = fc.shape[0]
    return pl.pallas_call(
        _edge_iter_last_body,
        grid=grid,
        in_specs=[
            pl.BlockSpec((BE, HID), row),
            pl.BlockSpec((BE, HID), row),
            pl.BlockSpec((BE, HID), row),
            pl.BlockSpec(gvT.shape, rep),
            pl.BlockSpec(geT.shape, rep),
            pl.BlockSpec(gb.shape, rep),
            pl.BlockSpec(wih.shape, rep),
            pl.BlockSpec(whh.shape, rep),
            pl.BlockSpec(bih.shape, rep),
            pl.BlockSpec(bhh.shape, rep),
            pl.BlockSpec(fc.shape, rep),
            pl.BlockSpec(fcb.shape, rep),
        ],
        out_specs=[
            pl.BlockSpec((BE, ncls), row),
            pl.BlockSpec((BE, HID), row),
            pl.BlockSpec((BE, HID), row),
        ],
        out_shape=[
            jax.ShapeDtypeStruct((HALF, ncls), jnp.float32),
            jax.ShapeDtypeStruct((HALF, HID), jnp.float32),
            jax.ShapeDtypeStruct((HALF, HID), jnp.float32),
        ],
    )(sub_v, obj_v, edge, gvT, geT, gb, wih, whh, bih, bhh, fc, fcb)


def _node_iter_body(s0_ref, s1_ref, vert_ref, wih_ref, whh_ref, bih_ref,
                    bhh_ref, out_ref):
    ctx = (s0_ref[:N_OBJ, :] + s0_ref[N_OBJ:, :]
           + s1_ref[:N_OBJ, :] + s1_ref[N_OBJ:, :])
    vert = vert_ref[...]
    gi = _dotT(ctx, wih_ref[...]) + bih_ref[...]
    gh = _dotT(vert, whh_ref[...]) + bhh_ref[...]
    out_ref[...] = _gru_tail(gi, gh, vert)


def _node_iter(s0, s1, vert, wih, whh, bih, bhh):
    return pl.pallas_call(
        _node_iter_body,
        out_shape=jax.ShapeDtypeStruct((N_OBJ, HID), jnp.float32),
    )(s0, s1, vert, wih, whh, bih, bhh)


def _node_iter_last_body(s0_ref, s1_ref, vert_ref, wih_ref, whh_ref, bih_ref,
                         bhh_ref, fc_ref, fcb_ref, out_ref):
    ctx = (s0_ref[:N_OBJ, :] + s0_ref[N_OBJ:, :]
           + s1_ref[:N_OBJ, :] + s1_ref[N_OBJ:, :])
    vert = vert_ref[...]
    gi = _dotT(ctx, wih_ref[...]) + bih_ref[...]
    gh = _dotT(vert, whh_ref[...]) + bhh_ref[...]
    vert_new = _gru_tail(gi, gh, vert)
    out_ref[...] = _dotT(vert_new, fc_ref[...]) + fcb_ref[...]


def _node_iter_last(s0, s1, vert, wih, whh, bih, bhh, fc, fcb):
    return pl.pallas_call(
        _node_iter_last_body,
        out_shape=jax.ShapeDtypeStruct((N_OBJ, fc.shape[0]), jnp.float32),
    )(s0, s1, vert, wih, whh, bih, bhh, fc, fcb)


_SC_MESH = plsc.VectorSubcoreMesh(core_axis_name="c", subcore_axis_name="s")


def _make_gather(off):
    @functools.partial(
        pl.kernel,
        mesh=_SC_MESH,
        out_type=[
            jax.ShapeDtypeStruct((HALF, HID), jnp.float32),
            jax.ShapeDtypeStruct((HALF, HID), jnp.float32),
        ],
        scratch_types=[
            pltpu.VMEM((_CH,), jnp.int32),
            pltpu.VMEM((_CH,), jnp.int32),
            pltpu.VMEM((_CH,), jnp.int32),
            pltpu.VMEM((_CH,), jnp.int32),
            pltpu.VMEM((_CH, HID), jnp.float32),
            pltpu.VMEM((_CH, HID), jnp.float32),
            pltpu.SemaphoreType.DMA,
            pltpu.SemaphoreType.DMA,
            pltpu.SemaphoreType.DMA,
            pltpu.SemaphoreType.DMA,
        ],
    )
    def gather(table_hbm, sidx_hbm, oidx_hbm, sub_out, obj_out,
               i0, i1, i2, i3, buf_a, buf_b, gs_a, gs_b, ws_a, ws_b):
        wid = lax.axis_index("s") * NC + lax.axis_index("c")
        bo = wid * EPW
        bi = off + bo
        pltpu.sync_copy(sidx_hbm.at[pl.ds(bi, _CH)], i0)
        g0 = pltpu.async_copy(table_hbm.at[i0], buf_a, gs_a)
        pltpu.sync_copy(sidx_hbm.at[pl.ds(bi + _CH, _CH)], i1)
        g1 = pltpu.async_copy(table_hbm.at[i1], buf_b, gs_b)
        pltpu.sync_copy(oidx_hbm.at[pl.ds(bi, _CH)], i2)
        pltpu.sync_copy(oidx_hbm.at[pl.ds(bi + _CH, _CH)], i3)
        g0.wait()
        w0 = pltpu.async_copy(buf_a, sub_out.at[pl.ds(bo, _CH)], ws_a)
        g1.wait()
        w1 = pltpu.async_copy(buf_b, sub_out.at[pl.ds(bo + _CH, _CH)], ws_b)
        w0.wait()
        g2 = pltpu.async_copy(table_hbm.at[i2], buf_a, gs_a)
        w1.wait()
        g3 = pltpu.async_copy(table_hbm.at[i3], buf_b, gs_b)
        g2.wait()
        w2 = pltpu.async_copy(buf_a, obj_out.at[pl.ds(bo, _CH)], ws_a)
        g3.wait()
        w3 = pltpu.async_copy(buf_b, obj_out.at[pl.ds(bo + _CH, _CH)], ws_b)
        w2.wait()
        w3.wait()

    return gather


def _make_scatter(off):
    @functools.partial(
        pl.kernel,
        mesh=_SC_MESH,
        out_type=jax.ShapeDtypeStruct((NC * N_OBJ, HID), jnp.float32),
        scratch_types=[
            pltpu.VMEM((_CH,), jnp.int32),
            pltpu.VMEM((_CH,), jnp.int32),
            pltpu.VMEM((_CH,), jnp.int32),
            pltpu.VMEM((_CH,), jnp.int32),
            pltpu.VMEM((_CH, HID), jnp.float32),
            pltpu.VMEM((_CH, HID), jnp.float32),
            pltpu.VMEM_SHARED((N_OBJ, HID), jnp.float32),
            pltpu.SemaphoreType.DMA,
            pltpu.SemaphoreType.DMA,
            pltpu.SemaphoreType.DMA,
        ],
    )
    def scatter(po_hbm, pi_hbm, sidx_hbm, oidx_hbm, zeros_hbm, out_hbm,
                i_s0, i_s1, i_o0, i_o1, buf_a, buf_b, acc, rs_a, rs_b, zs):
        c = lax.axis_index("c")
        s = lax.axis_index("s")
        z = pltpu.async_copy(zeros_hbm, acc.at[pl.ds(s * _ROWS_PER_TILE,
                                                     _ROWS_PER_TILE)], zs)
        bo = (c * NS + s) * EPW
        bi = off + bo
        pltpu.sync_copy(sidx_hbm.at[pl.ds(bi, _CH)], i_s0)
        pltpu.sync_copy(sidx_hbm.at[pl.ds(bi + _CH, _CH)], i_s1)
        pltpu.sync_copy(oidx_hbm.at[pl.ds(bi, _CH)], i_o0)
        pltpu.sync_copy(oidx_hbm.at[pl.ds(bi + _CH, _CH)], i_o1)
        r0 = pltpu.async_copy(po_hbm.at[pl.ds(bo, _CH)], buf_a, rs_a)
        r1 = pltpu.async_copy(po_hbm.at[pl.ds(bo + _CH, _CH)], buf_b, rs_b)
        z.wait()
        plsc.subcore_barrier()
        r0.wait()
        pltpu.sync_copy(buf_a, acc.at[i_s0], add=True)
        r2 = pltpu.async_copy(pi_hbm.at[pl.ds(bo, _CH)], buf_a, rs_a)
        r1.wait()
        pltpu.sync_copy(buf_b, acc.at[i_s1], add=True)
        r3 = pltpu.async_copy(pi_hbm.at[pl.ds(bo + _CH, _CH)], buf_b, rs_b)
        r2.wait()
        pltpu.sync_copy(buf_a, acc.at[i_o0], add=True)
        r3.wait()
        pltpu.sync_copy(buf_b, acc.at[i_o1], add=True)
        plsc.subcore_barrier()
        src = acc.at[pl.ds(s * _ROWS_PER_TILE, _ROWS_PER_TILE)]
        pltpu.sync_copy(src, out_hbm.at[pl.ds(c * N_OBJ + s * _ROWS_PER_TILE,
                                              _ROWS_PER_TILE)])

    return scatter


_gather_half = (_make_gather(0), _make_gather(HALF))
_scatter_half = (_make_scatter(0), _make_scatter(HALF))


def kernel(x, union_features, rel_pair_idxs, obj_unary_w, obj_unary_b,
           edge_unary_w, edge_unary_b, node_w_ih, node_w_hh, node_b_ih,
           node_b_hh, edge_w_ih, edge_w_hh, edge_b_ih, edge_b_hh, sub_w,
           sub_b, obj_w, obj_b, out_w, out_b, in_w, in_b, obj_fc_w,
           obj_fc_b, rel_fc_w, rel_fc_b):
    f32 = jnp.float32
    sidx = rel_pair_idxs[:, 0].astype(jnp.int32)
    oidx = rel_pair_idxs[:, 1].astype(jnp.int32)

    e_bih = edge_b_ih.reshape(1, -1)
    e_bhh = edge_b_hh.reshape(1, -1)
    n_bih = node_b_ih.reshape(1, -1)
    n_bhh = node_b_hh.reshape(1, -1)
    be = edge_unary_b.reshape(1, -1)
    bo = obj_unary_b.reshape(1, -1)
    obj_fcb = obj_fc_b.reshape(1, -1)
    rel_fcb = rel_fc_b.reshape(1, -1)

    gvT = jnp.zeros((HID, 128), f32).at[:, :4].set(
        jnp.stack([sub_w[0, :HID], obj_w[0, :HID],
                   out_w[0, :HID], in_w[0, :HID]], axis=1))
    geT = jnp.zeros((HID, 128), f32).at[:, :4].set(
        jnp.stack([sub_w[0, HID:], obj_w[0, HID:],
                   out_w[0, HID:], in_w[0, HID:]], axis=1))
    gb = jnp.zeros((1, 128), f32).at[0, :4].set(
        jnp.stack([sub_b[0], obj_b[0], out_b[0], in_b[0]]))

    vert = _vert_prologue(x, obj_unary_w, bo, node_w_ih, n_bih, n_bhh)
    e0 = _edge_prologue(union_features, edge_unary_w, be, edge_w_ih,
                        e_bih, e_bhh, 0)
    e1 = _edge_prologue(union_features, edge_unary_w, be, edge_w_ih,
                        e_bih, e_bhh, 1)
    edge = [e0, e1]

    zeros_tile = jnp.zeros((_ROWS_PER_TILE, HID), f32)

    for _ in range(2):
        scat = [None, None]
        new_edge = [None, None]
        for h in (0, 1):
            sub_v, obj_v = _gather_half[h](vert, sidx, oidx)
            new_edge[h], po, pi = _edge_iter(
                sub_v, obj_v, edge[h], gvT, geT, gb,
                edge_w_ih, edge_w_hh, e_bih, e_bhh)
            scat[h] = _scatter_half[h](po, pi, sidx, oidx, zeros_tile)
        vert = _node_iter(scat[0], scat[1], vert,
                          node_w_ih, node_w_hh, n_bih, n_bhh)
        edge = new_edge

    scat = [None, None]
    rel = [None, None]
    for h in (0, 1):
        sub_v, obj_v = _gather_half[h](vert, sidx, oidx)
        rel[h], po, pi = _edge_iter_last(
            sub_v, obj_v, edge[h], gvT, geT, gb,
            edge_w_ih, edge_w_hh, e_bih, e_bhh, rel_fc_w, rel_fcb)
        scat[h] = _scatter_half[h](po, pi, sidx, oidx, zeros_tile)
    obj_dists = _node_iter_last(scat[0], scat[1], vert, node_w_ih,
                                node_w_hh, n_bih, n_bhh, obj_fc_w, obj_fcb)
    rel_dists = jnp.concatenate(rel, axis=0)
    return (obj_dists, rel_dists)


# speedup vs baseline: 4.1686x; 1.0199x over previous
import functools

import jax
import jax.numpy as jnp
from jax import lax
from jax.experimental import pallas as pl
from jax.experimental.pallas import tpu as pltpu
from jax.experimental.pallas import tpu_sc as plsc

HID = 128
N_OBJ = 2048
N_REL = 16384
HALF = N_REL // 2
NUM_OBJ_CLS = 151
NUM_REL_CLS = 51

NC = 2
NS = 16
NW = NC * NS
EPW = HALF // NW
_CH = EPW // 2
_ROWS_PER_TILE = N_OBJ // NS

_DN_T = (((1,), (1,)), ((), ()))


def _sigmoid(x):
    return jax.nn.sigmoid(x)


def _gru_tail(gi, gh, h):
    r = _sigmoid(gi[:, :HID] + gh[:, :HID])
    z = _sigmoid(gi[:, HID:2 * HID] + gh[:, HID:2 * HID])
    n = jnp.tanh(gi[:, 2 * HID:] + r * gh[:, 2 * HID:])
    return (1.0 - z) * n + z * h


def _dotT(x, w):
    return lax.dot_general(x, w, _DN_T, preferred_element_type=jnp.float32)


def _edge_prologue_body(uf_ref, we_ref, be_ref, wih_ref, bih_ref, bhh_ref,
                        out_ref):
    rel = jnp.maximum(_dotT(uf_ref[...], we_ref[...]) + be_ref[...], 0.0)
    gi = _dotT(rel, wih_ref[...]) + bih_ref[...]
    bhh = bhh_ref[...]
    r = _sigmoid(gi[:, :HID] + bhh[:, :HID])
    z = _sigmoid(gi[:, HID:2 * HID] + bhh[:, HID:2 * HID])
    n = jnp.tanh(gi[:, 2 * HID:] + r * bhh[:, 2 * HID:])
    out_ref[...] = (1.0 - z) * n


def _edge_prologue(uf, we, be, wih, bih, bhh, half):
    BE = 1024
    nblk = HALF // BE
    off = half * nblk
    return pl.pallas_call(
        _edge_prologue_body,
        grid=(nblk,),
        in_specs=[
            pl.BlockSpec((BE, uf.shape[1]), lambda i: (i + off, 0)),
            pl.BlockSpec(we.shape, lambda i: (0, 0)),
            pl.BlockSpec(be.shape, lambda i: (0, 0)),
            pl.BlockSpec(wih.shape, lambda i: (0, 0)),
            pl.BlockSpec(bih.shape, lambda i: (0, 0)),
            pl.BlockSpec(bhh.shape, lambda i: (0, 0)),
        ],
        out_specs=pl.BlockSpec((BE, HID), lambda i: (i, 0)),
        out_shape=jax.ShapeDtypeStruct((HALF, HID), jnp.float32),
    )(uf, we, be, wih, bih, bhh)


def _vert_prologue_body(x_ref, wo_ref, bo_ref, wih_ref, bih_ref, bhh_ref,
                        out_ref):
    obj_rep = _dotT(x_ref[...], wo_ref[...]) + bo_ref[...]
    gi = _dotT(obj_rep, wih_ref[...]) + bih_ref[...]
    bhh = bhh_ref[...]
    r = _sigmoid(gi[:, :HID] + bhh[:, :HID])
    z = _sigmoid(gi[:, HID:2 * HID] + bhh[:, HID:2 * HID])
    n = jnp.tanh(gi[:, 2 * HID:] + r * bhh[:, 2 * HID:])
    out_ref[...] = (1.0 - z) * n


def _vert_prologue(x, wo, bo, wih, bih, bhh):
    return pl.pallas_call(
        _vert_prologue_body,
        out_shape=jax.ShapeDtypeStruct((N_OBJ, HID), jnp.float32),
    )(x, wo, bo, wih, bih, bhh)


def _edge_gates(sub, obj, edge, gvT, geT, gb):
    su = jnp.dot(sub, gvT, preferred_element_type=jnp.float32)
    ob = jnp.dot(obj, gvT, preferred_element_type=jnp.float32)
    ed = jnp.dot(edge, geT, preferred_element_type=jnp.float32)
    w_s = _sigmoid(su[:, 0:1] + ed[:, 0:1] + gb[0:1, 0:1])
    w_o = _sigmoid(ob[:, 1:2] + ed[:, 1:2] + gb[0:1, 1:2])
    g_out = _sigmoid(su[:, 2:3] + ed[:, 2:3] + gb[0:1, 2:3])
    g_in = _sigmoid(ob[:, 3:4] + ed[:, 3:4] + gb[0:1, 3:4])
    return w_s, w_o, g_out, g_in


def _edge_iter_body(sub_ref, obj_ref, edge_ref, gvT_ref, geT_ref, gb_ref,
                    wih_ref, whh_ref, bih_ref, bhh_ref,
                    eo_ref, po_ref, pi_ref):
    sub = sub_ref[...]
    obj = obj_ref[...]
    edge = edge_ref[...]
    w_s, w_o, g_out, g_in = _edge_gates(sub, obj, edge, gvT_ref[...],
                                        geT_ref[...], gb_ref[...])
    x_in = w_s * sub + w_o * obj
    gi = _dotT(x_in, wih_ref[...]) + bih_ref[...]
    gh = _dotT(edge, whh_ref[...]) + bhh_ref[...]
    eo_ref[...] = _gru_tail(gi, gh, edge)
    po_ref[...] = g_out * edge
    pi_ref[...] = g_in * edge


def _edge_iter(sub_v, obj_v, edge, gvT, geT, gb, wih, whh, bih, bhh):
    BE = 2048
    grid = (HALF // BE,)
    row = lambda i: (i, 0)
    rep = lambda i: (0, 0)
    return pl.pallas_call(
        _edge_iter_body,
        grid=grid,
        in_specs=[
            pl.BlockSpec((BE, HID), row),
            pl.BlockSpec((BE, HID), row),
            pl.BlockSpec((BE, HID), row),
            pl.BlockSpec(gvT.shape, rep),
            pl.BlockSpec(geT.shape, rep),
            pl.BlockSpec(gb.shape, rep),
            pl.BlockSpec(wih.shape, rep),
            pl.BlockSpec(whh.shape, rep),
            pl.BlockSpec(bih.shape, rep),
            pl.BlockSpec(bhh.shape, rep),
        ],
        out_specs=[
            pl.BlockSpec((BE, HID), row),
            pl.BlockSpec((BE, HID), row),
            pl.BlockSpec((BE, HID), row),
        ],
        out_shape=[
            jax.ShapeDtypeStruct((HALF, HID), jnp.float32),
            jax.ShapeDtypeStruct((HALF, HID), jnp.float32),
            jax.ShapeDtypeStruct((HALF, HID), jnp.float32),
        ],
    )(sub_v, obj_v, edge, gvT, geT, gb, wih, whh, bih, bhh)


def _edge_iter_last_body(sub_ref, obj_ref, edge_ref, gvT_ref, geT_ref, gb_ref,
                         wih_ref, whh_ref, bih_ref, bhh_ref,
                         fc_ref, fcb_ref, rd_ref, po_ref, pi_ref):
    sub = sub_ref[...]
    obj = obj_ref[...]
    edge = edge_ref[...]
    w_s, w_o, g_out, g_in = _edge_gates(sub, obj, edge, gvT_ref[...],
                                        geT_ref[...], gb_ref[...])
    x_in = w_s * sub + w_o * obj
    gi = _dotT(x_in, wih_ref[...]) + bih_ref[...]
    gh = _dotT(edge, whh_ref[...]) + bhh_ref[...]
    edge_new = _gru_tail(gi, gh, edge)
    rd_ref[...] = _dotT(edge_new, fc_ref[...]) + fcb_ref[...]
    po_ref[...] = g_out * edge
    pi_ref[...] = g_in * edge


def _edge_iter_last(sub_v, obj_v, edge, gvT, geT, gb, wih, whh, bih, bhh,
                    fc, fcb):
    BE = 2048
    grid = (HALF // BE,)
    row = lambda i: (i, 0)
    rep = lambda i: (0, 0)
    ncls = fc.shape[0]
    return pl.pallas_call(
        _edge_iter_last_body,
        grid=grid,
        in_specs=[
            pl.BlockSpec((BE, HID), row),
            pl.BlockSpec((BE, HID), row),
            pl.BlockSpec((BE, HID), row),
            pl.BlockSpec(gvT.shape, rep),
            pl.BlockSpec(geT.shape, rep),
            pl.BlockSpec(gb.shape, rep),
            pl.BlockSpec(wih.shape, rep),
            pl.BlockSpec(whh.shape, rep),
            pl.BlockSpec(bih.shape, rep),
            pl.BlockSpec(bhh.shape, rep),
            pl.BlockSpec(fc.shape, rep),
            pl.BlockSpec(fcb.shape, rep),
        ],
        out_specs=[
            pl.BlockSpec((BE, ncls), row),
            pl.BlockSpec((BE, HID), row),
            pl.BlockSpec((BE, HID), row),
        ],
        out_shape=[
            jax.ShapeDtypeStruct((HALF, ncls), jnp.float32),
            jax.ShapeDtypeStruct((HALF, HID), jnp.float32),
            jax.ShapeDtypeStruct((HALF, HID), jnp.float32),
        ],
    )(sub_v, obj_v, edge, gvT, geT, gb, wih, whh, bih, bhh, fc, fcb)


def _node_iter_body(s0_ref, s1_ref, vert_ref, wih_ref, whh_ref, bih_ref,
                    bhh_ref, out_ref):
    ctx = (s0_ref[:N_OBJ, :] + s0_ref[N_OBJ:, :]
           + s1_ref[:N_OBJ, :] + s1_ref[N_OBJ:, :])
    vert = vert_ref[...]
    gi = _dotT(ctx, wih_ref[...]) + bih_ref[...]
    gh = _dotT(vert, whh_ref[...]) + bhh_ref[...]
    out_ref[...] = _gru_tail(gi, gh, vert)


def _node_iter(s0, s1, vert, wih, whh, bih, bhh):
    return pl.pallas_call(
        _node_iter_body,
        out_shape=jax.ShapeDtypeStruct((N_OBJ, HID), jnp.float32),
    )(s0, s1, vert, wih, whh, bih, bhh)


def _node_iter_last_body(s0_ref, s1_ref, vert_ref, wih_ref, whh_ref, bih_ref,
                         bhh_ref, fc_ref, fcb_ref, out_ref):
    ctx = (s0_ref[:N_OBJ, :] + s0_ref[N_OBJ:, :]
           + s1_ref[:N_OBJ, :] + s1_ref[N_OBJ:, :])
    vert = vert_ref[...]
    gi = _dotT(ctx, wih_ref[...]) + bih_ref[...]
    gh = _dotT(vert, whh_ref[...]) + bhh_ref[...]
    vert_new = _gru_tail(gi, gh, vert)
    out_ref[...] = _dotT(vert_new, fc_ref[...]) + fcb_ref[...]


def _node_iter_last(s0, s1, vert, wih, whh, bih, bhh, fc, fcb):
    return pl.pallas_call(
        _node_iter_last_body,
        out_shape=jax.ShapeDtypeStruct((N_OBJ, fc.shape[0]), jnp.float32),
    )(s0, s1, vert, wih, whh, bih, bhh, fc, fcb)


_SC_MESH = plsc.VectorSubcoreMesh(core_axis_name="c", subcore_axis_name="s")


def _make_gather(off):
    @functools.partial(
        pl.kernel,
        mesh=_SC_MESH,
        out_type=[
            jax.ShapeDtypeStruct((HALF, HID), jnp.float32),
            jax.ShapeDtypeStruct((HALF, HID), jnp.float32),
        ],
        scratch_types=[
            pltpu.VMEM((EPW,), jnp.int32),
            pltpu.VMEM((EPW,), jnp.int32),
            pltpu.VMEM((EPW, HID), jnp.float32),
            pltpu.VMEM((EPW, HID), jnp.float32),
            pltpu.SemaphoreType.DMA,
            pltpu.SemaphoreType.DMA,
            pltpu.SemaphoreType.DMA,
            pltpu.SemaphoreType.DMA,
        ],
    )
    def gather(table_hbm, sidx_hbm, oidx_hbm, sub_out, obj_out,
               idx_s, idx_o, buf_a, buf_b, sa, sb, wa, wb):
        wid = lax.axis_index("s") * NC + lax.axis_index("c")
        bo = wid * EPW
        bi = off + bo
        ia = pltpu.async_copy(sidx_hbm.at[pl.ds(bi, EPW)], idx_s, sa)
        ib = pltpu.async_copy(oidx_hbm.at[pl.ds(bi, EPW)], idx_o, sb)
        ia.wait()
        g0 = pltpu.async_copy(table_hbm.at[idx_s], buf_a, sa)
        ib.wait()
        g1 = pltpu.async_copy(table_hbm.at[idx_o], buf_b, sb)
        g0.wait()
        w0 = pltpu.async_copy(buf_a, sub_out.at[pl.ds(bo, EPW)], wa)
        g1.wait()
        w1 = pltpu.async_copy(buf_b, obj_out.at[pl.ds(bo, EPW)], wb)
        w0.wait()
        w1.wait()

    return gather


def _make_scatter(off):
    @functools.partial(
        pl.kernel,
        mesh=_SC_MESH,
        out_type=jax.ShapeDtypeStruct((NC * N_OBJ, HID), jnp.float32),
        scratch_types=[
            pltpu.VMEM((EPW,), jnp.int32),
            pltpu.VMEM((EPW,), jnp.int32),
            pltpu.VMEM((EPW, HID), jnp.float32),
            pltpu.VMEM((EPW, HID), jnp.float32),
            pltpu.VMEM_SHARED((N_OBJ, HID), jnp.float32),
            pltpu.SemaphoreType.DMA,
            pltpu.SemaphoreType.DMA,
            pltpu.SemaphoreType.DMA,
        ],
    )
    def scatter(po_hbm, pi_hbm, sidx_hbm, oidx_hbm, zeros_hbm, out_hbm,
                idx_s, idx_o, buf_a, buf_b, acc, rs_a, rs_b, zs):
        c = lax.axis_index("c")
        s = lax.axis_index("s")
        z = pltpu.async_copy(zeros_hbm, acc.at[pl.ds(s * _ROWS_PER_TILE,
                                                     _ROWS_PER_TILE)], zs)
        bo = (c * NS + s) * EPW
        bi = off + bo
        ia = pltpu.async_copy(sidx_hbm.at[pl.ds(bi, EPW)], idx_s, rs_a)
        ib = pltpu.async_copy(oidx_hbm.at[pl.ds(bi, EPW)], idx_o, rs_b)
        ia.wait()
        ib.wait()
        r0 = pltpu.async_copy(po_hbm.at[pl.ds(bo, EPW)], buf_a, rs_a)
        r1 = pltpu.async_copy(pi_hbm.at[pl.ds(bo, EPW)], buf_b, rs_b)
        z.wait()
        plsc.subcore_barrier()
        r0.wait()
        pltpu.sync_copy(buf_a, acc.at[idx_s], add=True)
        r1.wait()
        pltpu.sync_copy(buf_b, acc.at[idx_o], add=True)
        plsc.subcore_barrier()
        src = acc.at[pl.ds(s * _ROWS_PER_TILE, _ROWS_PER_TILE)]
        pltpu.sync_copy(src, out_hbm.at[pl.ds(c * N_OBJ + s * _ROWS_PER_TILE,
                                              _ROWS_PER_TILE)])

    return scatter


_gather_half = (_make_gather(0), _make_gather(HALF))
_scatter_half = (_make_scatter(0), _make_scatter(HALF))


def kernel(x, union_features, rel_pair_idxs, obj_unary_w, obj_unary_b,
           edge_unary_w, edge_unary_b, node_w_ih, node_w_hh, node_b_ih,
           node_b_hh, edge_w_ih, edge_w_hh, edge_b_ih, edge_b_hh, sub_w,
           sub_b, obj_w, obj_b, out_w, out_b, in_w, in_b, obj_fc_w,
           obj_fc_b, rel_fc_w, rel_fc_b):
    f32 = jnp.float32
    sidx = rel_pair_idxs[:, 0].astype(jnp.int32)
    oidx = rel_pair_idxs[:, 1].astype(jnp.int32)

    e_bih = edge_b_ih.reshape(1, -1)
    e_bhh = edge_b_hh.reshape(1, -1)
    n_bih = node_b_ih.reshape(1, -1)
    n_bhh = node_b_hh.reshape(1, -1)
    be = edge_unary_b.reshape(1, -1)
    bo = obj_unary_b.reshape(1, -1)
    obj_fcb = obj_fc_b.reshape(1, -1)
    rel_fcb = rel_fc_b.reshape(1, -1)

    gvT = jnp.zeros((HID, 128), f32).at[:, :4].set(
        jnp.stack([sub_w[0, :HID], obj_w[0, :HID],
                   out_w[0, :HID], in_w[0, :HID]], axis=1))
    geT = jnp.zeros((HID, 128), f32).at[:, :4].set(
        jnp.stack([sub_w[0, HID:], obj_w[0, HID:],
                   out_w[0, HID:], in_w[0, HID:]], axis=1))
    gb = jnp.zeros((1, 128), f32).at[0, :4].set(
        jnp.stack([sub_b[0], obj_b[0], out_b[0], in_b[0]]))

    vert = _vert_prologue(x, obj_unary_w, bo, node_w_ih, n_bih, n_bhh)
    e0 = _edge_prologue(union_features, edge_unary_w, be, edge_w_ih,
                        e_bih, e_bhh, 0)
    e1 = _edge_prologue(union_features, edge_unary_w, be, edge_w_ih,
                        e_bih, e_bhh, 1)
    edge = [e0, e1]

    zeros_tile = jnp.zeros((_ROWS_PER_TILE, HID), f32)

    for _ in range(2):
        scat = [None, None]
        new_edge = [None, None]
        for h in (0, 1):
            sub_v, obj_v = _gather_half[h](vert, sidx, oidx)
            new_edge[h], po, pi = _edge_iter(
                sub_v, obj_v, edge[h], gvT, geT, gb,
                edge_w_ih, edge_w_hh, e_bih, e_bhh)
            scat[h] = _scatter_half[h](po, pi, sidx, oidx, zeros_tile)
        vert = _node_iter(scat[0], scat[1], vert,
                          node_w_ih, node_w_hh, n_bih, n_bhh)
        edge = new_edge

    scat = [None, None]
    rel = [None, None]
    for h in (0, 1):
        sub_v, obj_v = _gather_half[h](vert, sidx, oidx)
        rel[h], po, pi = _edge_iter_last(
            sub_v, obj_v, edge[h], gvT, geT, gb,
            edge_w_ih, edge_w_hh, e_bih, e_bhh, rel_fc_w, rel_fcb)
        scat[h] = _scatter_half[h](po, pi, sidx, oidx, zeros_tile)
    obj_dists = _node_iter_last(scat[0], scat[1], vert, node_w_ih,
                                node_w_hh, n_bih, n_bhh, obj_fc_w, obj_fcb)
    rel_dists = jnp.concatenate(rel, axis=0)
    return (obj_dists, rel_dists)
